# baseline (speedup 1.0000x reference)
import functools

import jax
import jax.numpy as jnp
from jax import lax
from jax.experimental import pallas as pl
from jax.experimental.pallas import tpu as pltpu
from jax.experimental.pallas import tpu_sc as plsc

N_ATOMS = 100000
N_SPECIES = 16
LANES = 16
NUM_WORKERS = 32
CHUNK = 3200


def _body(idx_hbm, scale_hbm, shift_hbm, s_hbm, b_hbm,
          idx_v, s_v, b_v, scale_v, shift_v,
          sem_t, sem_i0, sem_i1, sem_i2, sem_i3, sem_out):
    wid = lax.axis_index("s") * 2 + lax.axis_index("c")
    base = lax.min(wid * CHUNK, N_ATOMS - CHUNK)

    Q = CHUNK // 4
    cp_sc = pltpu.async_copy(scale_hbm, scale_v, sem_t)
    cp_sh = pltpu.async_copy(shift_hbm, shift_v, sem_t)
    cp_in = []
    for q, sem in enumerate((sem_i0, sem_i1, sem_i2, sem_i3)):
        cp_in.append(pltpu.async_copy(
            idx_hbm.at[pl.ds(base + q * Q, Q)],
            idx_v.at[pl.ds(q * Q, Q)], sem))
    cp_sc.wait()
    cp_sh.wait()
    scale_reg = scale_v[...]
    shift_reg = shift_v[...]

    dnums = lax.GatherDimensionNumbers(
        offset_dims=(), collapsed_slice_dims=(0,), start_index_map=(0,))

    def table_lookup(table_reg, idx16):
        return lax.gather(table_reg, idx16[:, None], dnums, slice_sizes=(1,),
                          mode=lax.GatherScatterMode.PROMISE_IN_BOUNDS)

    def lookup_quarter(lo):
        @plsc.parallel_loop(lo // LANES, (lo + Q) // LANES, unroll=10)
        def step(i):
            off = i * LANES
            idx16 = idx_v[pl.ds(off, LANES)]
            s_v[pl.ds(off, LANES)] = table_lookup(scale_reg, idx16)
            b_v[pl.ds(off, LANES)] = table_lookup(shift_reg, idx16)

    cp_out = []
    for q in range(4):
        cp_in[q].wait()
        lookup_quarter(q * Q)
        cp_out.append(pltpu.async_copy(
            s_v.at[pl.ds(q * Q, Q)], s_hbm.at[pl.ds(base + q * Q, Q)],
            sem_out))
        cp_out.append(pltpu.async_copy(
            b_v.at[pl.ds(q * Q, Q)], b_hbm.at[pl.ds(base + q * Q, Q)],
            sem_out))
    for cp in cp_out:
        cp.wait()


@jax.jit
def _rescale(x_2d, idx_i32, scale, shift):
    mesh = plsc.VectorSubcoreMesh(core_axis_name="c", subcore_axis_name="s")
    kfn = functools.partial(
        pl.kernel,
        out_type=(jax.ShapeDtypeStruct((N_ATOMS,), jnp.float32),
                  jax.ShapeDtypeStruct((N_ATOMS,), jnp.float32)),
        mesh=mesh,
        scratch_types=[
            pltpu.VMEM((CHUNK,), jnp.int32),
            pltpu.VMEM((CHUNK,), jnp.float32),
            pltpu.VMEM((CHUNK,), jnp.float32),
            pltpu.VMEM((N_SPECIES,), jnp.float32),
            pltpu.VMEM((N_SPECIES,), jnp.float32),
            pltpu.SemaphoreType.DMA,
            pltpu.SemaphoreType.DMA,
            pltpu.SemaphoreType.DMA,
            pltpu.SemaphoreType.DMA,
            pltpu.SemaphoreType.DMA,
            pltpu.SemaphoreType.DMA,
        ],
    )(_body)
    s_arr, b_arr = kfn(idx_i32, scale, shift)
    return x_2d * s_arr[:, None] + b_arr[:, None]


def kernel(x, indices, shift, scale):
    idx_i32 = indices.astype(jnp.int32)
    return _rescale(x, idx_i32, scale, shift)


# speedup vs baseline: 1.0834x; 1.0834x over previous
import functools

import jax
import jax.numpy as jnp
from jax import lax
from jax.experimental import pallas as pl
from jax.experimental.pallas import tpu as pltpu
from jax.experimental.pallas import tpu_sc as plsc

N_ATOMS = 100000
N_SPECIES = 16
LANES = 16
NUM_WORKERS = 16
CHUNK = 6400


def _body(idx_hbm, scale_hbm, shift_hbm, s_hbm, b_hbm,
          idx_v, s_v, b_v, scale_v, shift_v,
          sem_t, sem_i0, sem_i1, sem_i2, sem_i3, sem_out):
    wid = lax.axis_index("s") + lax.axis_index("c")
    base = lax.min(wid * CHUNK, N_ATOMS - CHUNK)

    Q = CHUNK // 4
    cp_sc = pltpu.async_copy(scale_hbm, scale_v, sem_t)
    cp_sh = pltpu.async_copy(shift_hbm, shift_v, sem_t)
    cp_in = []
    for q, sem in enumerate((sem_i0, sem_i1, sem_i2, sem_i3)):
        cp_in.append(pltpu.async_copy(
            idx_hbm.at[pl.ds(base + q * Q, Q)],
            idx_v.at[pl.ds(q * Q, Q)], sem))
    cp_sc.wait()
    cp_sh.wait()
    scale_reg = scale_v[...]
    shift_reg = shift_v[...]

    dnums = lax.GatherDimensionNumbers(
        offset_dims=(), collapsed_slice_dims=(0,), start_index_map=(0,))

    def table_lookup(table_reg, idx16):
        return lax.gather(table_reg, idx16[:, None], dnums, slice_sizes=(1,),
                          mode=lax.GatherScatterMode.PROMISE_IN_BOUNDS)

    def lookup_quarter(lo):
        @plsc.parallel_loop(lo // LANES, (lo + Q) // LANES, unroll=10)
        def step(i):
            off = i * LANES
            idx16 = idx_v[pl.ds(off, LANES)]
            s_v[pl.ds(off, LANES)] = table_lookup(scale_reg, idx16)
            b_v[pl.ds(off, LANES)] = table_lookup(shift_reg, idx16)

    cp_out = []
    for q in range(4):
        cp_in[q].wait()
        lookup_quarter(q * Q)
        cp_out.append(pltpu.async_copy(
            s_v.at[pl.ds(q * Q, Q)], s_hbm.at[pl.ds(base + q * Q, Q)],
            sem_out))
        cp_out.append(pltpu.async_copy(
            b_v.at[pl.ds(q * Q, Q)], b_hbm.at[pl.ds(base + q * Q, Q)],
            sem_out))
    for cp in cp_out:
        cp.wait()


@jax.jit
def _rescale(x_2d, idx_i32, scale, shift):
    mesh = plsc.VectorSubcoreMesh(
        core_axis_name="c", subcore_axis_name="s", num_cores=1)
    kfn = functools.partial(
        pl.kernel,
        out_type=(jax.ShapeDtypeStruct((N_ATOMS,), jnp.float32),
                  jax.ShapeDtypeStruct((N_ATOMS,), jnp.float32)),
        mesh=mesh,
        scratch_types=[
            pltpu.VMEM((CHUNK,), jnp.int32),
            pltpu.VMEM((CHUNK,), jnp.float32),
            pltpu.VMEM((CHUNK,), jnp.float32),
            pltpu.VMEM((N_SPECIES,), jnp.float32),
            pltpu.VMEM((N_SPECIES,), jnp.float32),
            pltpu.SemaphoreType.DMA,
            pltpu.SemaphoreType.DMA,
            pltpu.SemaphoreType.DMA,
            pltpu.SemaphoreType.DMA,
            pltpu.SemaphoreType.DMA,
            pltpu.SemaphoreType.DMA,
        ],
    )(_body)
    s_arr, b_arr = kfn(idx_i32, scale, shift)
    return x_2d * s_arr[:, None] + b_arr[:, None]


def kernel(x, indices, shift, scale):
    idx_i32 = indices.astype(jnp.int32)
    return _rescale(x, idx_i32, scale, shift)
